# baseline (speedup 1.0000x reference)
import functools

import jax
import jax.numpy as jnp
from jax import lax
from jax.experimental import pallas as pl
from jax.experimental.pallas import tpu as pltpu
from jax.experimental.pallas import tpu_sc as plsc

HIDDEN = 64
ROWB = 2 * HIDDEN
CHUNK = 512
LANES = 16


def _embed(idx, tpad, s_count, b_count):
    n = idx.shape[0]
    info = plsc.get_sparse_core_info()
    nw = info.num_cores * info.num_subcores
    blocks_per_s = b_count // CHUNK
    u_per_w = (n // CHUNK) // nw
    mesh = plsc.VectorSubcoreMesh(core_axis_name="c", subcore_axis_name="s")

    @functools.partial(
        pl.kernel,
        mesh=mesh,
        out_type=jax.ShapeDtypeStruct((s_count, HIDDEN, b_count), jnp.float32),
        scratch_types=[
            pltpu.VMEM((CHUNK,), jnp.int32),
            pltpu.VMEM((CHUNK, ROWB), jnp.float32),
            pltpu.VMEM((HIDDEN, CHUNK), jnp.float32),
            pltpu.SemaphoreType.DMA,
        ],
        compiler_params=pltpu.CompilerParams(
            use_tc_tiling_on_sc=True, needs_layout_passes=False
        ),
    )
    def emb(idx_hbm, tab_hbm, out_hbm, idx_v, rows_v, stage_v, sem):
        wid = lax.axis_index("s") * info.num_cores + lax.axis_index("c")
        u0 = wid * u_per_w
        lanes = lax.iota(jnp.int32, LANES)

        def unit_body(u, carry):
            uid = u0 + u
            s = uid // blocks_per_s
            blk = uid % blocks_per_s
            off = pl.multiple_of(uid * CHUNK, CHUNK)
            pltpu.sync_copy(idx_hbm.at[pl.ds(off, CHUNK)], idx_v)
            pltpu.async_copy(tab_hbm.at[idx_v], rows_v, sem).wait()

            def m_body(m, c2):
                tok = m * LANES + lanes
                for f in range(HIDDEN):
                    col = jnp.full((LANES,), f, jnp.int32)
                    vals = plsc.load_gather(rows_v, [tok, col])
                    stage_v[f, pl.ds(m * LANES, LANES)] = vals
                return c2

            lax.fori_loop(0, CHUNK // LANES, m_body, 0)
            b0 = pl.multiple_of(blk * CHUNK, CHUNK)
            pltpu.sync_copy(stage_v, out_hbm.at[s, :, pl.ds(b0, CHUNK)])
            return carry

        lax.fori_loop(0, u_per_w, unit_body, 0)

    return emb(idx, tpad)


def kernel(tokens, embedding_table):
    b, s = tokens.shape
    idx = tokens.T.reshape(b * s)
    tpad = jnp.pad(embedding_table, ((0, 0), (0, ROWB - HIDDEN)))
    out3 = _embed(idx, tpad, s, b)
    return (tokens, out3.transpose(2, 0, 1))


# speedup vs baseline: 1.2384x; 1.2384x over previous
import functools

import jax
import jax.numpy as jnp
from jax import lax
from jax.experimental import pallas as pl
from jax.experimental.pallas import tpu as pltpu
from jax.experimental.pallas import tpu_sc as plsc

HIDDEN = 64
ROWB = 2 * HIDDEN
CHUNK = 512
LANES = 16


def _embed(idx, tpad, s_count, b_count):
    n = idx.shape[0]
    info = plsc.get_sparse_core_info()
    nw = info.num_cores * info.num_subcores
    blocks_per_s = b_count // CHUNK
    u_per_w = (n // CHUNK) // nw
    mesh = plsc.VectorSubcoreMesh(core_axis_name="c", subcore_axis_name="s")

    @functools.partial(
        pl.kernel,
        mesh=mesh,
        out_type=jax.ShapeDtypeStruct((s_count, HIDDEN, b_count), jnp.float32),
        scratch_types=[
            pltpu.VMEM((CHUNK,), jnp.int32),
            pltpu.VMEM((CHUNK, ROWB), jnp.float32),
            pltpu.VMEM((HIDDEN, CHUNK), jnp.float32),
            pltpu.SemaphoreType.DMA,
        ],
        compiler_params=pltpu.CompilerParams(
            use_tc_tiling_on_sc=True,
            needs_layout_passes=False,
            disable_bounds_checks=True,
        ),
    )
    def emb(idx_hbm, tab_hbm, out_hbm, idx_v, rows_v, stage_v, sem):
        wid = lax.axis_index("s") * info.num_cores + lax.axis_index("c")
        u0 = wid * u_per_w
        lanes = lax.iota(jnp.int32, LANES)

        def unit_body(u, carry):
            uid = u0 + u
            s = uid // blocks_per_s
            blk = uid % blocks_per_s
            off = pl.multiple_of(uid * CHUNK, CHUNK)
            pltpu.sync_copy(idx_hbm.at[pl.ds(off, CHUNK)], idx_v)
            pltpu.async_copy(tab_hbm.at[idx_v], rows_v, sem).wait()

            def m_body(m, c2):
                tok = m * LANES + lanes
                for w in range(0, HIDDEN, 8):
                    vals = [
                        plsc.load_gather(
                            rows_v, [tok, jnp.full((LANES,), w + k, jnp.int32)]
                        )
                        for k in range(8)
                    ]
                    for k in range(8):
                        stage_v[w + k, pl.ds(m * LANES, LANES)] = vals[k]
                return c2

            lax.fori_loop(0, CHUNK // LANES, m_body, 0)
            b0 = pl.multiple_of(blk * CHUNK, CHUNK)
            pltpu.sync_copy(stage_v, out_hbm.at[s, :, pl.ds(b0, CHUNK)])
            return carry

        lax.fori_loop(0, u_per_w, unit_body, 0)

    return emb(idx, tpad)


def kernel(tokens, embedding_table):
    b, s = tokens.shape
    idx = tokens.T.reshape(b * s)
    tpad = jnp.pad(embedding_table, ((0, 0), (0, ROWB - HIDDEN)))
    out3 = _embed(idx, tpad, s, b)
    return (tokens, out3.transpose(2, 0, 1))


# speedup vs baseline: 1.3689x; 1.1054x over previous
import functools

import jax
import jax.numpy as jnp
from jax import lax
from jax.experimental import pallas as pl
from jax.experimental.pallas import tpu as pltpu
from jax.experimental.pallas import tpu_sc as plsc

HIDDEN = 64
ROWB = 2 * HIDDEN
CHUNK = 256
LANES = 16
WAVE = 8


def _embed(idx, tpad, s_count, b_count):
    n = idx.shape[0]
    info = plsc.get_sparse_core_info()
    nw = info.num_cores * info.num_subcores
    blocks_per_s = b_count // CHUNK
    u_per_w = (n // CHUNK) // nw
    n_pairs = u_per_w // 2
    mesh = plsc.VectorSubcoreMesh(core_axis_name="c", subcore_axis_name="s")

    @functools.partial(
        pl.kernel,
        mesh=mesh,
        out_type=jax.ShapeDtypeStruct((s_count, HIDDEN, b_count), jnp.float32),
        scratch_types=[
            pltpu.VMEM((u_per_w * CHUNK,), jnp.int32),
            pltpu.VMEM((2, CHUNK, ROWB), jnp.float32),
            pltpu.VMEM((2, HIDDEN, CHUNK), jnp.float32),
            pltpu.SemaphoreType.DMA,
            pltpu.SemaphoreType.DMA,
        ],
        compiler_params=pltpu.CompilerParams(
            use_tc_tiling_on_sc=True,
            needs_layout_passes=False,
            disable_bounds_checks=True,
        ),
    )
    def emb(idx_hbm, tab_hbm, out_hbm, idx_all, rows_v, stage_v, g0, g1, *_):
        wid = lax.axis_index("s") * info.num_cores + lax.axis_index("c")
        u0 = wid * u_per_w
        tok0 = pl.multiple_of(u0 * CHUNK, CHUNK)
        pltpu.sync_copy(idx_hbm.at[pl.ds(tok0, u_per_w * CHUNK)], idx_all)
        gsem = (g0, g1)
        lanes = lax.iota(jnp.int32, LANES)

        def idx_slice(ul):
            o = pl.multiple_of(ul * CHUNK, CHUNK)
            return idx_all.at[pl.ds(o, CHUNK)]

        def fire(ul, sl):
            pltpu.async_copy(tab_hbm.at[idx_slice(ul)], rows_v.at[sl], gsem[sl])

        def gwait(ul, sl):
            pltpu.make_async_copy(
                tab_hbm.at[idx_slice(ul)], rows_v.at[sl], gsem[sl]
            ).wait()

        def transpose_store(ul, sl):
            rows = rows_v.at[sl]
            stage = stage_v.at[sl]

            def m_body(m, c2):
                tok = m * LANES + lanes
                for w in range(0, HIDDEN, WAVE):
                    vals = [
                        plsc.load_gather(
                            rows, [tok, jnp.full((LANES,), w + k, jnp.int32)]
                        )
                        for k in range(WAVE)
                    ]
                    for k in range(WAVE):
                        stage[w + k, pl.ds(m * LANES, LANES)] = vals[k]
                return c2

            lax.fori_loop(0, CHUNK // LANES, m_body, 0)
            uid = u0 + ul
            s = uid // blocks_per_s
            b0 = pl.multiple_of((uid % blocks_per_s) * CHUNK, CHUNK)
            pltpu.sync_copy(stage, out_hbm.at[s, :, pl.ds(b0, CHUNK)])

        fire(0, 0)

        def pair_body(g, carry):
            ua = 2 * g
            gwait(ua, 0)
            fire(ua + 1, 1)
            transpose_store(ua, 0)
            gwait(ua + 1, 1)
            fire(jnp.minimum(ua + 2, u_per_w - 1), 0)
            transpose_store(ua + 1, 1)
            return carry

        lax.fori_loop(0, n_pairs, pair_body, 0)
        gwait(u_per_w - 1, 0)

    return emb(idx, tpad)


def kernel(tokens, embedding_table):
    b, s = tokens.shape
    idx = tokens.T.reshape(b * s)
    tpad = jnp.pad(embedding_table, ((0, 0), (0, ROWB - HIDDEN)))
    out3 = _embed(idx, tpad, s, b)
    return (tokens, out3.transpose(2, 0, 1))
